# baseline (speedup 1.0000x reference)
import jax
import jax.numpy as jnp
from jax import lax
from jax.experimental import pallas as pl
from jax.experimental.pallas import tpu as pltpu
from jax.experimental.pallas import tpu_sc as plsc

N_NUM = 13
N_CAT = 26
VOCAB = 100000
D = 64
B = 4096
N_TOK = 1 + N_NUM + N_CAT

NUM_CORES = 2
NUM_SUBCORES = 16
NW = NUM_CORES * NUM_SUBCORES

W_IDS = 4096
WIN_BASE = [k * W_IDS for k in range(24)] + [24 * W_IDS, 99968]
WIN_W = [W_IDS] * 24 + [1664, 128]
NWIN = len(WIN_BASE)
NBND = 32

N_CGRP = N_CAT * (D // 8)
N_DGRP = (1 + N_NUM) * (D // 8)

IOFF_BND = N_CAT * B
FOFF_PD = N_NUM * B


def _tokenizer_kernel(fprep_hbm, iprep_hbm, tt_hbm, aux_hbm, out_hbm,
                      wbuf, obuf, skeys_v, xn_v, bnd_v, prm_v,
                      ssem, osem, psem, dsem):
    core = lax.axis_index("c")
    sub = lax.axis_index("s")
    wid = sub * NUM_CORES + core

    lanes = lax.iota(jnp.int32, 16)

    ndense = jnp.where(wid < 16, 3, 4)

    def dense_body(j, _):
        h = (31 - wid) + 32 * j
        t = h // 8
        dg = h % 8
        tm1 = jnp.maximum(t - 1, 0)
        xoff = pl.multiple_of(tm1 * B, 8)
        pltpu.async_copy(fprep_hbm.at[pl.ds(xoff, B)], xn_v, dsem)
        poff = pl.multiple_of(FOFF_PD + h * 16, 8)
        pltpu.async_copy(fprep_hbm.at[pl.ds(poff, 16)], prm_v, psem)
        pltpu.make_async_copy(fprep_hbm.at[pl.ds(xoff, B)], xn_v, dsem).wait()
        pltpu.make_async_copy(fprep_hbm.at[pl.ds(poff, 16)], prm_v,
                              psem).wait()
        pv = prm_v[...]

        for dd in range(8):
            s_mul = pv[dd]
            s_add = pv[8 + dd]

            def row_body(v, _):
                sl = pl.ds(v * 16, 16)
                obuf[dd, sl] = xn_v[sl] * s_mul + s_add
                return 0

            lax.fori_loop(0, B // 16, row_body, 0)

        doff = pl.multiple_of(dg * 8, 8)
        pltpu.async_copy(obuf, out_hbm.at[t, pl.ds(doff, 8)], osem)
        pltpu.make_async_copy(obuf, out_hbm.at[t, pl.ds(doff, 8)], osem).wait()
        return 0

    lax.fori_loop(0, ndense, dense_body, 0)

    ncat = jnp.where(wid < 16, 7, 6)

    def prime_group(i):
        g = wid + 32 * i
        r0 = pl.multiple_of(g * 8, 8)
        pltpu.async_copy(tt_hbm.at[pl.ds(r0, 8), pl.ds(0, W_IDS)],
                         wbuf.at[0], ssem)

    prime_group(jnp.int32(0))

    def cat_body(i, _):
        g = wid + 32 * i
        f = g // 8
        dg = g % 8
        r0 = pl.multiple_of(g * 8, 8)

        koff = pl.multiple_of(f * B, 8)
        boff = pl.multiple_of(IOFF_BND + f * NBND, 8)
        pltpu.async_copy(iprep_hbm.at[pl.ds(koff, B)], skeys_v, psem)
        pltpu.async_copy(iprep_hbm.at[pl.ds(boff, NBND)], bnd_v, psem)
        pltpu.make_async_copy(iprep_hbm.at[pl.ds(koff, B)], skeys_v,
                              psem).wait()
        pltpu.make_async_copy(iprep_hbm.at[pl.ds(boff, NBND)], bnd_v,
                              psem).wait()
        bndlo = bnd_v[pl.ds(0, 16)]
        bndhi = bnd_v[pl.ds(16, 16)]

        def bnd_at(k):
            return bndlo[k] if k < 16 else bndhi[k - 16]

        def win_src(k):
            if k == NWIN - 1:
                return aux_hbm.at[pl.ds(r0, 8)], WIN_W[k]
            return (tt_hbm.at[pl.ds(r0, 8), pl.ds(WIN_BASE[k], WIN_W[k])],
                    WIN_W[k])

        for k in range(NWIN):
            src, wk = win_src(k)
            buf = k % 2
            if k + 1 < NWIN:
                src1, wk1 = win_src(k + 1)
                pltpu.async_copy(src1, wbuf.at[1 - buf, :, pl.ds(0, wk1)],
                                 ssem)
            else:
                @pl.when(i + 1 < ncat)
                def _():
                    prime_group(i + 1)

            pltpu.make_async_copy(src, wbuf.at[buf, :, pl.ds(0, wk)],
                                  ssem).wait()

            lo = bnd_at(k)
            hi = bnd_at(k + 1)
            lo16 = lax.shift_right_logical(lo, 4)
            nblk = jnp.maximum(
                lax.shift_right_logical(hi + 15, 4) - lo16, 0)
            nblk = jnp.where(hi > lo, nblk, 0)

            def blk_body(jb, _):
                base = (lo16 + jb) * 16
                kv = skeys_v[pl.ds(base, 16)]
                lid = base + lanes
                msk = (lid >= lo) & (lid < hi)
                vv = lax.shift_right_logical(kv, 12) - WIN_BASE[k]
                vv = jnp.clip(vv, 0, jnp.int32(wk - 1))
                pos = lax.bitwise_and(kv, 4095)
                for dd in range(8):
                    dvec = jnp.full((16,), dd, jnp.int32)
                    gval = plsc.load_gather(wbuf.at[buf], [dvec, vv])
                    plsc.store_scatter(obuf, [dvec, pos], gval, mask=msk)
                return 0

            lax.fori_loop(0, nblk, blk_body, 0)

        t = 1 + N_NUM + f
        doff = pl.multiple_of(dg * 8, 8)
        pltpu.async_copy(obuf, out_hbm.at[t, pl.ds(doff, 8)], osem)
        pltpu.make_async_copy(obuf, out_hbm.at[t, pl.ds(doff, 8)], osem).wait()
        return 0

    lax.fori_loop(0, ncat, cat_body, 0)


@jax.jit
def kernel(x_num, x_cat, num_weight, num_bias, cat_tables, cls_token):
    tt = cat_tables.transpose(0, 2, 1).reshape(N_CAT * D, VOCAB)
    aux = jnp.pad(tt[:, WIN_BASE[-1]:], ((0, 0), (0, 96)))
    xcat_t = x_cat.astype(jnp.int32).T
    keys = xcat_t * B + jnp.arange(B, dtype=jnp.int32)[None, :]
    skeys = jnp.sort(keys, axis=1)
    bvals = jnp.array(WIN_BASE + [VOCAB], dtype=jnp.int32)
    bnds = jnp.sum(xcat_t[:, :, None] < bvals[None, None, :],
                   axis=1, dtype=jnp.int32)
    bnds = jnp.pad(bnds, ((0, 0), (0, NBND - bnds.shape[1])))
    iprep = jnp.concatenate([skeys.reshape(-1), bnds.reshape(-1)])

    cls = cls_token.reshape(D)
    tq = jnp.arange(N_DGRP, dtype=jnp.int32) // 8
    dd = (jnp.arange(N_DGRP, dtype=jnp.int32) % 8)[:, None] * 8 \
        + jnp.arange(8, dtype=jnp.int32)[None, :]
    tm1 = jnp.maximum(tq - 1, 0)
    mul = jnp.where(tq[:, None] == 0, 0.0, num_weight[tm1[:, None], dd])
    add = jnp.where(tq[:, None] == 0, cls[dd], num_bias[tm1[:, None], dd])
    pdense = jnp.concatenate([mul, add], axis=1).reshape(-1)
    fprep = jnp.concatenate([x_num.T.reshape(-1), pdense])

    mesh = plsc.VectorSubcoreMesh(core_axis_name="c", subcore_axis_name="s",
                                  num_cores=NUM_CORES,
                                  num_subcores=NUM_SUBCORES)
    run = pl.kernel(
        _tokenizer_kernel,
        out_type=jax.ShapeDtypeStruct((N_TOK, D, B), jnp.float32),
        mesh=mesh,
        scratch_types=[
            pltpu.VMEM((2, 8, W_IDS), jnp.float32),
            pltpu.VMEM((8, B), jnp.float32),
            pltpu.VMEM((B,), jnp.int32),
            pltpu.VMEM((B,), jnp.float32),
            pltpu.VMEM((NBND,), jnp.int32),
            pltpu.VMEM((16,), jnp.float32),
            pltpu.SemaphoreType.DMA,
            pltpu.SemaphoreType.DMA,
            pltpu.SemaphoreType.DMA,
            pltpu.SemaphoreType.DMA,
        ],
        compiler_params=pltpu.CompilerParams(needs_layout_passes=False),
    )
    out_t = run(fprep, iprep, tt, aux)
    return out_t.transpose(2, 0, 1)


# speedup vs baseline: 1.0362x; 1.0362x over previous
import jax
import jax.numpy as jnp
from jax import lax
from jax.experimental import pallas as pl
from jax.experimental.pallas import tpu as pltpu
from jax.experimental.pallas import tpu_sc as plsc

N_NUM = 13
N_CAT = 26
VOCAB = 100000
D = 64
B = 4096
N_TOK = 1 + N_NUM + N_CAT

NUM_CORES = 2
NUM_SUBCORES = 16
NW = NUM_CORES * NUM_SUBCORES

W_IDS = 4096
WIN_BASE = [k * W_IDS for k in range(24)] + [24 * W_IDS, 99968]
WIN_W = [W_IDS] * 24 + [1664, 128]
NWIN = len(WIN_BASE)
NBND = 32

N_CGRP = N_CAT * (D // 8)
N_DGRP = (1 + N_NUM) * (D // 8)

def _tokenizer_kernel(xn1d_hbm, pd_hbm, skeys1d_hbm, bnds1d_hbm,
                      tt_hbm, aux_hbm, out_hbm,
                      wbuf, obuf, skeys_v, xn_v, bnd_v, prm_v,
                      ssem, osem, psem, dsem):
    core = lax.axis_index("c")
    sub = lax.axis_index("s")
    wid = sub * NUM_CORES + core

    lanes = lax.iota(jnp.int32, 16)

    ndense = jnp.where(wid < 16, 3, 4)

    def dense_body(j, _):
        h = (31 - wid) + 32 * j
        t = h // 8
        dg = h % 8
        tm1 = jnp.maximum(t - 1, 0)
        xoff = pl.multiple_of(tm1 * B, 8)
        pltpu.async_copy(xn1d_hbm.at[pl.ds(xoff, B)], xn_v, dsem)
        poff = pl.multiple_of(h * 16, 8)
        pltpu.async_copy(pd_hbm.at[pl.ds(poff, 16)], prm_v, psem)
        pltpu.make_async_copy(xn1d_hbm.at[pl.ds(xoff, B)], xn_v, dsem).wait()
        pltpu.make_async_copy(pd_hbm.at[pl.ds(poff, 16)], prm_v,
                              psem).wait()
        pv = prm_v[...]

        for dd in range(8):
            s_mul = pv[dd]
            s_add = pv[8 + dd]

            def row_body(v, _):
                sl = pl.ds(v * 16, 16)
                obuf[dd, sl] = xn_v[sl] * s_mul + s_add
                return 0

            lax.fori_loop(0, B // 16, row_body, 0)

        doff = pl.multiple_of(dg * 8, 8)
        pltpu.async_copy(obuf, out_hbm.at[t, pl.ds(doff, 8)], osem)
        pltpu.make_async_copy(obuf, out_hbm.at[t, pl.ds(doff, 8)], osem).wait()
        return 0

    lax.fori_loop(0, ndense, dense_body, 0)

    ncat = jnp.where(wid < 16, 7, 6)

    def prime_group(i):
        g = wid + 32 * i
        r0 = pl.multiple_of(g * 8, 8)
        pltpu.async_copy(tt_hbm.at[pl.ds(r0, 8), pl.ds(0, W_IDS)],
                         wbuf.at[0], ssem)

    prime_group(jnp.int32(0))

    def cat_body(i, _):
        g = wid + 32 * i
        f = g // 8
        dg = g % 8
        r0 = pl.multiple_of(g * 8, 8)

        koff = pl.multiple_of(f * B, 8)
        boff = pl.multiple_of(f * NBND, 8)
        pltpu.async_copy(skeys1d_hbm.at[pl.ds(koff, B)], skeys_v, psem)
        pltpu.async_copy(bnds1d_hbm.at[pl.ds(boff, NBND)], bnd_v, psem)
        pltpu.make_async_copy(skeys1d_hbm.at[pl.ds(koff, B)], skeys_v,
                              psem).wait()
        pltpu.make_async_copy(bnds1d_hbm.at[pl.ds(boff, NBND)], bnd_v,
                              psem).wait()
        bndlo = bnd_v[pl.ds(0, 16)]
        bndhi = bnd_v[pl.ds(16, 16)]

        def bnd_at(k):
            return bndlo[k] if k < 16 else bndhi[k - 16]

        def win_src(k):
            if k == NWIN - 1:
                return aux_hbm.at[pl.ds(r0, 8)], WIN_W[k]
            return (tt_hbm.at[pl.ds(r0, 8), pl.ds(WIN_BASE[k], WIN_W[k])],
                    WIN_W[k])

        for k in range(NWIN):
            src, wk = win_src(k)
            buf = k % 2
            if k + 1 < NWIN:
                src1, wk1 = win_src(k + 1)
                pltpu.async_copy(src1, wbuf.at[1 - buf, :, pl.ds(0, wk1)],
                                 ssem)
            else:
                @pl.when(i + 1 < ncat)
                def _():
                    prime_group(i + 1)

            pltpu.make_async_copy(src, wbuf.at[buf, :, pl.ds(0, wk)],
                                  ssem).wait()

            lo = bnd_at(k)
            hi = bnd_at(k + 1)
            lo16 = lax.shift_right_logical(lo, 4)
            nblk = jnp.maximum(
                lax.shift_right_logical(hi + 15, 4) - lo16, 0)
            nblk = jnp.where(hi > lo, nblk, 0)

            def blk_body(jb, _):
                base = (lo16 + jb) * 16
                kv = skeys_v[pl.ds(base, 16)]
                lid = base + lanes
                msk = (lid >= lo) & (lid < hi)
                vv = lax.shift_right_logical(kv, 12) - WIN_BASE[k]
                vv = jnp.clip(vv, 0, jnp.int32(wk - 1))
                pos = lax.bitwise_and(kv, 4095)
                for dd in range(8):
                    dvec = jnp.full((16,), dd, jnp.int32)
                    gval = plsc.load_gather(wbuf.at[buf], [dvec, vv])
                    plsc.store_scatter(obuf, [dvec, pos], gval, mask=msk)
                return 0

            lax.fori_loop(0, nblk, blk_body, 0)

        t = 1 + N_NUM + f
        doff = pl.multiple_of(dg * 8, 8)
        pltpu.async_copy(obuf, out_hbm.at[t, pl.ds(doff, 8)], osem)
        pltpu.make_async_copy(obuf, out_hbm.at[t, pl.ds(doff, 8)], osem).wait()
        return 0

    lax.fori_loop(0, ncat, cat_body, 0)


@jax.jit
def kernel(x_num, x_cat, num_weight, num_bias, cat_tables, cls_token):
    tt = cat_tables.transpose(0, 2, 1).reshape(N_CAT * D, VOCAB)
    aux = jnp.pad(tt[:, WIN_BASE[-1]:], ((0, 0), (0, 96)))
    xcat_t = x_cat.astype(jnp.int32).T
    keys = xcat_t * B + jnp.arange(B, dtype=jnp.int32)[None, :]
    skeys = jnp.sort(keys, axis=1)
    bvals = jnp.array(WIN_BASE + [VOCAB], dtype=jnp.int32)
    bnds = jnp.sum(xcat_t[:, :, None] < bvals[None, None, :],
                   axis=1, dtype=jnp.int32)
    bnds = jnp.pad(bnds, ((0, 0), (0, NBND - bnds.shape[1])))

    mul = jnp.concatenate([jnp.zeros((8, 8), jnp.float32),
                           num_weight.reshape(N_NUM * 8, 8)])
    add = jnp.concatenate([cls_token.reshape(8, 8),
                           num_bias.reshape(N_NUM * 8, 8)])
    pdense = jnp.concatenate([mul, add], axis=1).reshape(-1)

    mesh = plsc.VectorSubcoreMesh(core_axis_name="c", subcore_axis_name="s",
                                  num_cores=NUM_CORES,
                                  num_subcores=NUM_SUBCORES)
    run = pl.kernel(
        _tokenizer_kernel,
        out_type=jax.ShapeDtypeStruct((N_TOK, D, B), jnp.float32),
        mesh=mesh,
        scratch_types=[
            pltpu.VMEM((2, 8, W_IDS), jnp.float32),
            pltpu.VMEM((8, B), jnp.float32),
            pltpu.VMEM((B,), jnp.int32),
            pltpu.VMEM((B,), jnp.float32),
            pltpu.VMEM((NBND,), jnp.int32),
            pltpu.VMEM((16,), jnp.float32),
            pltpu.SemaphoreType.DMA,
            pltpu.SemaphoreType.DMA,
            pltpu.SemaphoreType.DMA,
            pltpu.SemaphoreType.DMA,
        ],
        compiler_params=pltpu.CompilerParams(needs_layout_passes=False),
    )
    out_t = run(x_num.T.reshape(-1), pdense, skeys.reshape(-1),
                bnds.reshape(-1), tt, aux)
    return out_t.transpose(2, 0, 1)


# speedup vs baseline: 1.0613x; 1.0243x over previous
import jax
import jax.numpy as jnp
from jax import lax
from jax.experimental import pallas as pl
from jax.experimental.pallas import tpu as pltpu
from jax.experimental.pallas import tpu_sc as plsc

N_NUM = 13
N_CAT = 26
VOCAB = 100000
D = 64
B = 4096
N_TOK = 1 + N_NUM + N_CAT

NUM_CORES = 2
NUM_SUBCORES = 16
NW = NUM_CORES * NUM_SUBCORES

W_IDS = 3712
WIN_BASE = [k * W_IDS for k in range(26)] + [26 * W_IDS, 99968]
WIN_W = [W_IDS] * 26 + [99968 - 26 * W_IDS, 128]
NWIN = len(WIN_BASE)
NBND = 32

N_CGRP = N_CAT * (D // 8)
N_DGRP = (1 + N_NUM) * (D // 8)

def _tokenizer_kernel(xn1d_hbm, pd_hbm, skeys1d_hbm, bnds1d_hbm,
                      tt_hbm, aux_hbm, out_hbm,
                      wbuf, obuf, skeys_v, bnd_v, prm_v,
                      ssem, osem, psem, dsem):
    core = lax.axis_index("c")
    sub = lax.axis_index("s")
    wid = sub * NUM_CORES + core

    lanes = lax.iota(jnp.int32, 16)
    ncat = jnp.where(wid < 16, 7, 6)

    def stage_keys(i):
        g = wid + 32 * i
        f = g // 8
        koff = pl.multiple_of(f * B, 8)
        boff = pl.multiple_of(f * NBND, 8)
        pltpu.async_copy(skeys1d_hbm.at[pl.ds(koff, B)], skeys_v, psem)
        pltpu.async_copy(bnds1d_hbm.at[pl.ds(boff, NBND)], bnd_v, psem)

    def wait_keys(i):
        g = wid + 32 * i
        f = g // 8
        koff = pl.multiple_of(f * B, 8)
        boff = pl.multiple_of(f * NBND, 8)
        pltpu.make_async_copy(skeys1d_hbm.at[pl.ds(koff, B)], skeys_v,
                              psem).wait()
        pltpu.make_async_copy(bnds1d_hbm.at[pl.ds(boff, NBND)], bnd_v,
                              psem).wait()

    def prime_group(i):
        g = wid + 32 * i
        r0 = pl.multiple_of(g * 8, 8)
        pltpu.async_copy(tt_hbm.at[pl.ds(r0, 8), pl.ds(0, W_IDS)],
                         wbuf.at[0], ssem)

    prime_group(jnp.int32(0))
    stage_keys(jnp.int32(0))

    ndense = jnp.where(wid < 16, 3, 4)

    def dense_body(j, _):
        h = (31 - wid) + 32 * j
        t = h // 8
        dg = h % 8
        tm1 = jnp.maximum(t - 1, 0)
        xoff = pl.multiple_of(tm1 * B, 8)
        xn_v = obuf.at[1, 0]
        pltpu.async_copy(xn1d_hbm.at[pl.ds(xoff, B)], xn_v, dsem)
        poff = pl.multiple_of(h * 16, 8)
        pltpu.async_copy(pd_hbm.at[pl.ds(poff, 16)], prm_v, psem)
        pltpu.make_async_copy(xn1d_hbm.at[pl.ds(xoff, B)], xn_v, dsem).wait()
        pltpu.make_async_copy(pd_hbm.at[pl.ds(poff, 16)], prm_v,
                              psem).wait()
        pv = prm_v[...]

        for dd in range(8):
            s_mul = pv[dd]
            s_add = pv[8 + dd]

            def row_body(v, _):
                sl = pl.ds(v * 16, 16)
                obuf[0, dd, sl] = obuf[1, 0, sl] * s_mul + s_add
                return 0

            lax.fori_loop(0, B // 16, row_body, 0)

        doff = pl.multiple_of(dg * 8, 8)
        pltpu.async_copy(obuf.at[0], out_hbm.at[t, pl.ds(doff, 8)], osem)
        pltpu.make_async_copy(obuf.at[0], out_hbm.at[t, pl.ds(doff, 8)],
                              osem).wait()
        return 0

    lax.fori_loop(0, ndense, dense_body, 0)

    def group_work(i, par):
        ob = obuf.at[par]
        osem_p = osem if par == 0 else dsem
        g = wid + 32 * i
        f = g // 8
        dg = g % 8
        r0 = pl.multiple_of(g * 8, 8)

        wait_keys(i)
        bndlo = bnd_v[pl.ds(0, 16)]
        bndhi = bnd_v[pl.ds(16, 16)]

        @pl.when(i >= 2)
        def _():
            pltpu.make_async_copy(
                ob, out_hbm.at[1 + N_NUM, pl.ds(0, 8)], osem_p).wait()

        def bnd_at(k):
            return bndlo[k] if k < 16 else bndhi[k - 16]

        def win_src(k):
            if k == NWIN - 1:
                return aux_hbm.at[pl.ds(r0, 8)], WIN_W[k]
            return (tt_hbm.at[pl.ds(r0, 8), pl.ds(WIN_BASE[k], WIN_W[k])],
                    WIN_W[k])

        for k in range(NWIN):
            src, wk = win_src(k)
            buf = k % 2
            if k + 1 < NWIN:
                src1, wk1 = win_src(k + 1)
                pltpu.async_copy(src1, wbuf.at[1 - buf, :, pl.ds(0, wk1)],
                                 ssem)
            else:
                @pl.when(i + 1 < ncat)
                def _():
                    prime_group(i + 1)

            pltpu.make_async_copy(src, wbuf.at[buf, :, pl.ds(0, wk)],
                                  ssem).wait()

            lo = bnd_at(k)
            hi = bnd_at(k + 1)
            lo16 = lax.shift_right_logical(lo, 4)
            nblk = jnp.maximum(
                lax.shift_right_logical(hi + 15, 4) - lo16, 0)
            nblk = jnp.where(hi > lo, nblk, 0)

            def blk_body(jb, _):
                base = (lo16 + jb) * 16
                kv = skeys_v[pl.ds(base, 16)]
                lid = base + lanes
                msk = (lid >= lo) & (lid < hi)
                vv = lax.shift_right_logical(kv, 12) - WIN_BASE[k]
                vv = jnp.clip(vv, 0, jnp.int32(wk - 1))
                pos = lax.bitwise_and(kv, 4095)
                for dd in range(8):
                    dvec = jnp.full((16,), dd, jnp.int32)
                    gval = plsc.load_gather(wbuf.at[buf], [dvec, vv])
                    plsc.store_scatter(ob, [dvec, pos], gval, mask=msk)
                return 0

            lax.fori_loop(0, nblk, blk_body, 0)

        @pl.when(i + 1 < ncat)
        def _():
            stage_keys(i + 1)

        t = 1 + N_NUM + f
        doff = pl.multiple_of(dg * 8, 8)
        pltpu.async_copy(ob, out_hbm.at[t, pl.ds(doff, 8)], osem_p)

    def cat2_body(j, _):
        for par in range(2):
            i = 2 * j + par

            @pl.when(i < ncat)
            def _():
                group_work(i, par)
        return 0

    lax.fori_loop(0, (ncat + 1) // 2, cat2_body, 0)

    pltpu.make_async_copy(obuf.at[0], out_hbm.at[1 + N_NUM, pl.ds(0, 8)],
                          osem).wait()
    pltpu.make_async_copy(obuf.at[1], out_hbm.at[1 + N_NUM, pl.ds(0, 8)],
                          dsem).wait()


@jax.jit
def kernel(x_num, x_cat, num_weight, num_bias, cat_tables, cls_token):
    tt = cat_tables.transpose(0, 2, 1).reshape(N_CAT * D, VOCAB)
    aux = jnp.pad(tt[:, WIN_BASE[-1]:], ((0, 0), (0, 96)))
    xcat_t = x_cat.astype(jnp.int32).T
    keys = xcat_t * B + jnp.arange(B, dtype=jnp.int32)[None, :]
    skeys = jnp.sort(keys, axis=1)
    bvals = jnp.array(WIN_BASE + [VOCAB], dtype=jnp.int32)
    bnds = jnp.sum(xcat_t[:, :, None] < bvals[None, None, :],
                   axis=1, dtype=jnp.int32)
    bnds = jnp.pad(bnds, ((0, 0), (0, NBND - bnds.shape[1])))

    mul = jnp.concatenate([jnp.zeros((8, 8), jnp.float32),
                           num_weight.reshape(N_NUM * 8, 8)])
    add = jnp.concatenate([cls_token.reshape(8, 8),
                           num_bias.reshape(N_NUM * 8, 8)])
    pdense = jnp.concatenate([mul, add], axis=1).reshape(-1)

    mesh = plsc.VectorSubcoreMesh(core_axis_name="c", subcore_axis_name="s",
                                  num_cores=NUM_CORES,
                                  num_subcores=NUM_SUBCORES)
    run = pl.kernel(
        _tokenizer_kernel,
        out_type=jax.ShapeDtypeStruct((N_TOK, D, B), jnp.float32),
        mesh=mesh,
        scratch_types=[
            pltpu.VMEM((2, 8, W_IDS), jnp.float32),
            pltpu.VMEM((2, 8, B), jnp.float32),
            pltpu.VMEM((B,), jnp.int32),
            pltpu.VMEM((NBND,), jnp.int32),
            pltpu.VMEM((16,), jnp.float32),
            pltpu.SemaphoreType.DMA,
            pltpu.SemaphoreType.DMA,
            pltpu.SemaphoreType.DMA,
            pltpu.SemaphoreType.DMA,
        ],
        compiler_params=pltpu.CompilerParams(needs_layout_passes=False),
    )
    out_t = run(x_num.T.reshape(-1), pdense, skeys.reshape(-1),
                bnds.reshape(-1), tt, aux)
    return out_t.transpose(2, 0, 1)


# speedup vs baseline: 1.0626x; 1.0012x over previous
import jax
import jax.numpy as jnp
from jax import lax
from jax.experimental import pallas as pl
from jax.experimental.pallas import tpu as pltpu
from jax.experimental.pallas import tpu_sc as plsc

N_NUM = 13
N_CAT = 26
VOCAB = 100000
D = 64
B = 4096
N_TOK = 1 + N_NUM + N_CAT

NUM_CORES = 2
NUM_SUBCORES = 16
NW = NUM_CORES * NUM_SUBCORES

W_IDS = 3712
WIN_BASE = [k * W_IDS for k in range(26)] + [26 * W_IDS, 99968]
WIN_W = [W_IDS] * 26 + [99968 - 26 * W_IDS, 128]
NWIN = len(WIN_BASE)
NBND = 32

N_CGRP = N_CAT * (D // 8)
N_DGRP = (1 + N_NUM) * (D // 8)

def _tokenizer_kernel(xn1d_hbm, pd_hbm, skeys1d_hbm, bnds1d_hbm,
                      tt_hbm, aux_hbm, out_hbm,
                      wbuf, obuf, skeys_v, bnd_v, prm_v,
                      ssem, osem, psem, dsem, ksem):
    core = lax.axis_index("c")
    sub = lax.axis_index("s")
    wid = sub * NUM_CORES + core

    lanes = lax.iota(jnp.int32, 16)
    ncat = jnp.where(wid < 16, 7, 6)

    def stage_keys(i):
        g = wid + 32 * i
        f = g // 8
        koff = pl.multiple_of(f * B, 8)
        boff = pl.multiple_of(f * NBND, 8)
        pltpu.async_copy(skeys1d_hbm.at[pl.ds(koff, B)], skeys_v, ksem)
        pltpu.async_copy(bnds1d_hbm.at[pl.ds(boff, NBND)], bnd_v, ksem)

    def wait_keys(i):
        g = wid + 32 * i
        f = g // 8
        koff = pl.multiple_of(f * B, 8)
        boff = pl.multiple_of(f * NBND, 8)
        pltpu.make_async_copy(skeys1d_hbm.at[pl.ds(koff, B)], skeys_v,
                              ksem).wait()
        pltpu.make_async_copy(bnds1d_hbm.at[pl.ds(boff, NBND)], bnd_v,
                              ksem).wait()

    def prime_group(i):
        g = wid + 32 * i
        r0 = pl.multiple_of(g * 8, 8)
        pltpu.async_copy(tt_hbm.at[pl.ds(r0, 8), pl.ds(0, W_IDS)],
                         wbuf.at[0], ssem)

    prime_group(jnp.int32(0))
    stage_keys(jnp.int32(0))

    ndense = jnp.where(wid < 16, 3, 4)

    def dense_body(j, _):
        h = (31 - wid) + 32 * j
        t = h // 8
        dg = h % 8
        tm1 = jnp.maximum(t - 1, 0)
        xoff = pl.multiple_of(tm1 * B, 8)
        xn_v = obuf.at[1, 0]
        pltpu.async_copy(xn1d_hbm.at[pl.ds(xoff, B)], xn_v, dsem)
        poff = pl.multiple_of(h * 16, 8)
        pltpu.async_copy(pd_hbm.at[pl.ds(poff, 16)], prm_v, psem)
        pltpu.make_async_copy(xn1d_hbm.at[pl.ds(xoff, B)], xn_v, dsem).wait()
        pltpu.make_async_copy(pd_hbm.at[pl.ds(poff, 16)], prm_v,
                              psem).wait()
        pv = prm_v[...]

        for dd in range(8):
            s_mul = pv[dd]
            s_add = pv[8 + dd]

            def row_body(v, _):
                sl = pl.ds(v * 16, 16)
                obuf[0, dd, sl] = obuf[1, 0, sl] * s_mul + s_add
                return 0

            lax.fori_loop(0, B // 16, row_body, 0)

        doff = pl.multiple_of(dg * 8, 8)
        pltpu.async_copy(obuf.at[0], out_hbm.at[t, pl.ds(doff, 8)], osem)
        pltpu.make_async_copy(obuf.at[0], out_hbm.at[t, pl.ds(doff, 8)],
                              osem).wait()
        return 0

    lax.fori_loop(0, ndense, dense_body, 0)

    def group_work(i, par):
        ob = obuf.at[par]
        osem_p = osem if par == 0 else dsem
        g = wid + 32 * i
        f = g // 8
        dg = g % 8
        r0 = pl.multiple_of(g * 8, 8)

        wait_keys(i)
        bndlo = bnd_v[pl.ds(0, 16)]
        bndhi = bnd_v[pl.ds(16, 16)]

        @pl.when(i >= 2)
        def _():
            pltpu.make_async_copy(
                ob, out_hbm.at[1 + N_NUM, pl.ds(0, 8)], osem_p).wait()

        def bnd_at(k):
            return bndlo[k] if k < 16 else bndhi[k - 16]

        def win_src(k):
            if k == NWIN - 1:
                return aux_hbm.at[pl.ds(r0, 8)], WIN_W[k]
            return (tt_hbm.at[pl.ds(r0, 8), pl.ds(WIN_BASE[k], WIN_W[k])],
                    WIN_W[k])

        for k in range(NWIN):
            src, wk = win_src(k)
            buf = k % 2
            if k + 1 < NWIN:
                src1, wk1 = win_src(k + 1)
                pltpu.async_copy(src1, wbuf.at[1 - buf, :, pl.ds(0, wk1)],
                                 ssem)
            else:
                @pl.when(i + 1 < ncat)
                def _():
                    prime_group(i + 1)

            pltpu.make_async_copy(src, wbuf.at[buf, :, pl.ds(0, wk)],
                                  ssem).wait()

            lo = bnd_at(k)
            hi = bnd_at(k + 1)
            lo16 = lax.shift_right_logical(lo, 4)
            nblk = jnp.maximum(
                lax.shift_right_logical(hi + 15, 4) - lo16, 0)
            nblk = jnp.where(hi > lo, nblk, 0)

            def blk_body(jb, _):
                base = (lo16 + jb) * 16
                kv = skeys_v[pl.ds(base, 16)]
                lid = base + lanes
                msk = (lid >= lo) & (lid < hi)
                vv = lax.shift_right_logical(kv, 12) - WIN_BASE[k]
                vv = jnp.clip(vv, 0, jnp.int32(wk - 1))
                pos = lax.bitwise_and(kv, 4095)
                for dd in range(8):
                    dvec = jnp.full((16,), dd, jnp.int32)
                    gval = plsc.load_gather(wbuf.at[buf], [dvec, vv])
                    plsc.store_scatter(ob, [dvec, pos], gval, mask=msk)
                return 0

            lax.fori_loop(0, nblk, blk_body, 0)

        @pl.when(i + 1 < ncat)
        def _():
            stage_keys(i + 1)

        t = 1 + N_NUM + f
        doff = pl.multiple_of(dg * 8, 8)
        pltpu.async_copy(ob, out_hbm.at[t, pl.ds(doff, 8)], osem_p)

    def cat2_body(j, _):
        for par in range(2):
            i = 2 * j + par

            @pl.when(i < ncat)
            def _():
                group_work(i, par)
        return 0

    lax.fori_loop(0, (ncat + 1) // 2, cat2_body, 0)

    pltpu.make_async_copy(obuf.at[0], out_hbm.at[1 + N_NUM, pl.ds(0, 8)],
                          osem).wait()
    pltpu.make_async_copy(obuf.at[1], out_hbm.at[1 + N_NUM, pl.ds(0, 8)],
                          dsem).wait()


@jax.jit
def kernel(x_num, x_cat, num_weight, num_bias, cat_tables, cls_token):
    tt = cat_tables.transpose(0, 2, 1).reshape(N_CAT * D, VOCAB)
    aux = jnp.pad(tt[:, WIN_BASE[-1]:], ((0, 0), (0, 96)))
    xcat_t = x_cat.astype(jnp.int32).T
    keys = xcat_t * B + jnp.arange(B, dtype=jnp.int32)[None, :]
    skeys = jnp.sort(keys, axis=1)
    bvals = jnp.array(WIN_BASE + [VOCAB], dtype=jnp.int32)
    bnds = jnp.sum(xcat_t[:, :, None] < bvals[None, None, :],
                   axis=1, dtype=jnp.int32)
    bnds = jnp.pad(bnds, ((0, 0), (0, NBND - bnds.shape[1])))

    mul = jnp.concatenate([jnp.zeros((8, 8), jnp.float32),
                           num_weight.reshape(N_NUM * 8, 8)])
    add = jnp.concatenate([cls_token.reshape(8, 8),
                           num_bias.reshape(N_NUM * 8, 8)])
    pdense = jnp.concatenate([mul, add], axis=1).reshape(-1)

    mesh = plsc.VectorSubcoreMesh(core_axis_name="c", subcore_axis_name="s",
                                  num_cores=NUM_CORES,
                                  num_subcores=NUM_SUBCORES)
    run = pl.kernel(
        _tokenizer_kernel,
        out_type=jax.ShapeDtypeStruct((N_TOK, D, B), jnp.float32),
        mesh=mesh,
        scratch_types=[
            pltpu.VMEM((2, 8, W_IDS), jnp.float32),
            pltpu.VMEM((2, 8, B), jnp.float32),
            pltpu.VMEM((B,), jnp.int32),
            pltpu.VMEM((NBND,), jnp.int32),
            pltpu.VMEM((16,), jnp.float32),
            pltpu.SemaphoreType.DMA,
            pltpu.SemaphoreType.DMA,
            pltpu.SemaphoreType.DMA,
            pltpu.SemaphoreType.DMA,
            pltpu.SemaphoreType.DMA,
        ],
        compiler_params=pltpu.CompilerParams(needs_layout_passes=False),
    )
    out_t = run(x_num.T.reshape(-1), pdense, skeys.reshape(-1),
                bnds.reshape(-1), tt, aux)
    return out_t.transpose(2, 0, 1)


# speedup vs baseline: 1.1092x; 1.0439x over previous
import jax
import jax.numpy as jnp
from jax import lax
from jax.experimental import pallas as pl
from jax.experimental.pallas import tpu as pltpu
from jax.experimental.pallas import tpu_sc as plsc

N_NUM = 13
N_CAT = 26
VOCAB = 100000
D = 64
B = 4096
N_TOK = 1 + N_NUM + N_CAT

NUM_CORES = 2
NUM_SUBCORES = 16
NW = NUM_CORES * NUM_SUBCORES

W_IDS = 3712
WIN_BASE = [k * W_IDS for k in range(26)] + [26 * W_IDS, 99968]
WIN_W = [W_IDS] * 26 + [99968 - 26 * W_IDS, 128]
NWIN = len(WIN_BASE)
NBND = 32

N_CGRP = N_CAT * (D // 8)
N_DGRP = (1 + N_NUM) * (D // 8)

def _tokenizer_kernel(xn1d_hbm, pd_hbm, skeys1d_hbm, bnds1d_hbm,
                      tt_hbm, aux_hbm, out_hbm,
                      wbuf, obuf, skeys_v, bnd_v, prm_v,
                      ssem, osem, psem, dsem, ksem):
    core = lax.axis_index("c")
    sub = lax.axis_index("s")
    wid = sub * NUM_CORES + core

    lanes = lax.iota(jnp.int32, 16)
    ncat = jnp.where(wid < 16, 7, 6)

    def stage_keys(i):
        g = wid + 32 * i
        f = g // 8
        koff = pl.multiple_of(f * B, 8)
        boff = pl.multiple_of(f * NBND, 8)
        pltpu.async_copy(skeys1d_hbm.at[pl.ds(koff, B)], skeys_v, ksem)
        pltpu.async_copy(bnds1d_hbm.at[pl.ds(boff, NBND)], bnd_v, ksem)

    def wait_keys(i):
        g = wid + 32 * i
        f = g // 8
        koff = pl.multiple_of(f * B, 8)
        boff = pl.multiple_of(f * NBND, 8)
        pltpu.make_async_copy(skeys1d_hbm.at[pl.ds(koff, B)], skeys_v,
                              ksem).wait()
        pltpu.make_async_copy(bnds1d_hbm.at[pl.ds(boff, NBND)], bnd_v,
                              ksem).wait()

    def prime_group(i):
        g = wid + 32 * i
        r0 = pl.multiple_of(g * 8, 8)
        pltpu.async_copy(tt_hbm.at[pl.ds(r0, 8), pl.ds(0, W_IDS)],
                         wbuf.at[0], ssem)

    prime_group(jnp.int32(0))
    stage_keys(jnp.int32(0))

    ndense = jnp.where(wid < 16, 3, 4)

    def dense_body(j, _):
        h = (31 - wid) + 32 * j
        t = h // 8
        dg = h % 8
        tm1 = jnp.maximum(t - 1, 0)
        xoff = pl.multiple_of(tm1 * B, 8)
        xn_v = obuf.at[1, 0]
        pltpu.async_copy(xn1d_hbm.at[pl.ds(xoff, B)], xn_v, dsem)
        poff = pl.multiple_of(h * 16, 8)
        pltpu.async_copy(pd_hbm.at[pl.ds(poff, 16)], prm_v, psem)
        pltpu.make_async_copy(xn1d_hbm.at[pl.ds(xoff, B)], xn_v, dsem).wait()
        pltpu.make_async_copy(pd_hbm.at[pl.ds(poff, 16)], prm_v,
                              psem).wait()
        pv = prm_v[...]
        muls = [pv[dd] for dd in range(8)]
        adds = [pv[8 + dd] for dd in range(8)]

        def row_body(v, _):
            sl = pl.ds(v * 16, 16)
            xv = obuf[1, 0, sl]
            for dd in range(8):
                obuf[0, dd, sl] = xv * muls[dd] + adds[dd]
            return 0

        lax.fori_loop(0, B // 16, row_body, 0, unroll=2)

        doff = pl.multiple_of(dg * 8, 8)
        pltpu.async_copy(obuf.at[0], out_hbm.at[t, pl.ds(doff, 8)], osem)
        pltpu.make_async_copy(obuf.at[0], out_hbm.at[t, pl.ds(doff, 8)],
                              osem).wait()
        return 0

    lax.fori_loop(0, ndense, dense_body, 0)

    def group_work(i, par):
        ob = obuf.at[par]
        osem_p = osem if par == 0 else dsem
        g = wid + 32 * i
        f = g // 8
        dg = g % 8
        r0 = pl.multiple_of(g * 8, 8)

        wait_keys(i)
        bndlo = bnd_v[pl.ds(0, 16)]
        bndhi = bnd_v[pl.ds(16, 16)]

        @pl.when(i >= 2)
        def _():
            pltpu.make_async_copy(
                ob, out_hbm.at[1 + N_NUM, pl.ds(0, 8)], osem_p).wait()

        def bnd_at(k):
            return bndlo[k] if k < 16 else bndhi[k - 16]

        def win_src(k):
            if k == NWIN - 1:
                return aux_hbm.at[pl.ds(r0, 8)], WIN_W[k]
            return (tt_hbm.at[pl.ds(r0, 8), pl.ds(WIN_BASE[k], WIN_W[k])],
                    WIN_W[k])

        for k in range(NWIN):
            src, wk = win_src(k)
            buf = k % 2
            if k + 1 < NWIN:
                src1, wk1 = win_src(k + 1)
                pltpu.async_copy(src1, wbuf.at[1 - buf, :, pl.ds(0, wk1)],
                                 ssem)
            else:
                @pl.when(i + 1 < ncat)
                def _():
                    prime_group(i + 1)

            pltpu.make_async_copy(src, wbuf.at[buf, :, pl.ds(0, wk)],
                                  ssem).wait()

            lo = bnd_at(k)
            hi = bnd_at(k + 1)
            lo16 = lax.shift_right_logical(lo, 4)
            nblk = jnp.maximum(
                lax.shift_right_logical(hi + 15, 4) - lo16, 0)
            nblk = jnp.where(hi > lo, nblk, 0)

            def blk_body(jb, _):
                base = (lo16 + jb) * 16
                kv = skeys_v[pl.ds(base, 16)]
                lid = base + lanes
                msk = (lid >= lo) & (lid < hi)
                vv = lax.shift_right_logical(kv, 12) - WIN_BASE[k]
                vv = jnp.clip(vv, 0, jnp.int32(wk - 1))
                pos = lax.bitwise_and(kv, 4095)
                for dd in range(8):
                    dvec = jnp.full((16,), dd, jnp.int32)
                    gval = plsc.load_gather(wbuf.at[buf], [dvec, vv])
                    plsc.store_scatter(ob, [dvec, pos], gval, mask=msk)
                return 0

            lax.fori_loop(0, nblk, blk_body, 0)

        @pl.when(i + 1 < ncat)
        def _():
            stage_keys(i + 1)

        t = 1 + N_NUM + f
        doff = pl.multiple_of(dg * 8, 8)
        pltpu.async_copy(ob, out_hbm.at[t, pl.ds(doff, 8)], osem_p)

    def cat2_body(j, _):
        for par in range(2):
            i = 2 * j + par

            @pl.when(i < ncat)
            def _():
                group_work(i, par)
        return 0

    lax.fori_loop(0, (ncat + 1) // 2, cat2_body, 0)

    pltpu.make_async_copy(obuf.at[0], out_hbm.at[1 + N_NUM, pl.ds(0, 8)],
                          osem).wait()
    pltpu.make_async_copy(obuf.at[1], out_hbm.at[1 + N_NUM, pl.ds(0, 8)],
                          dsem).wait()


@jax.jit
def kernel(x_num, x_cat, num_weight, num_bias, cat_tables, cls_token):
    tt = cat_tables.transpose(0, 2, 1).reshape(N_CAT * D, VOCAB)
    aux = jnp.pad(tt[:, WIN_BASE[-1]:], ((0, 0), (0, 96)))
    xcat_t = x_cat.astype(jnp.int32).T
    keys = xcat_t * B + jnp.arange(B, dtype=jnp.int32)[None, :]
    skeys = jnp.sort(keys, axis=1)
    bvals = jnp.array(WIN_BASE + [VOCAB], dtype=jnp.int32)
    bnds = jnp.sum(xcat_t[:, :, None] < bvals[None, None, :],
                   axis=1, dtype=jnp.int32)
    bnds = jnp.pad(bnds, ((0, 0), (0, NBND - bnds.shape[1])))

    mul = jnp.concatenate([jnp.zeros((8, 8), jnp.float32),
                           num_weight.reshape(N_NUM * 8, 8)])
    add = jnp.concatenate([cls_token.reshape(8, 8),
                           num_bias.reshape(N_NUM * 8, 8)])
    pdense = jnp.concatenate([mul, add], axis=1).reshape(-1)

    mesh = plsc.VectorSubcoreMesh(core_axis_name="c", subcore_axis_name="s",
                                  num_cores=NUM_CORES,
                                  num_subcores=NUM_SUBCORES)
    run = pl.kernel(
        _tokenizer_kernel,
        out_type=jax.ShapeDtypeStruct((N_TOK, D, B), jnp.float32),
        mesh=mesh,
        scratch_types=[
            pltpu.VMEM((2, 8, W_IDS), jnp.float32),
            pltpu.VMEM((2, 8, B), jnp.float32),
            pltpu.VMEM((B,), jnp.int32),
            pltpu.VMEM((NBND,), jnp.int32),
            pltpu.VMEM((16,), jnp.float32),
            pltpu.SemaphoreType.DMA,
            pltpu.SemaphoreType.DMA,
            pltpu.SemaphoreType.DMA,
            pltpu.SemaphoreType.DMA,
            pltpu.SemaphoreType.DMA,
        ],
        compiler_params=pltpu.CompilerParams(needs_layout_passes=False),
    )
    out_t = run(x_num.T.reshape(-1), pdense, skeys.reshape(-1),
                bnds.reshape(-1), tt, aux)
    return out_t.transpose(2, 0, 1)
